# baseline (speedup 1.0000x reference)
import jax
import jax.numpy as jnp
from jax import lax
from jax.experimental import pallas as pl
from jax.experimental.pallas import tpu as pltpu
from jax.experimental.pallas import tpu_sc as plsc

_NC = 2
_NS = 16
_NW = _NC * _NS

_BATCH = 4096
_HIST = 200
_V = 1000000
_D = 64
_W = 2 * _D

_B = _BATCH * _HIST
_IW = 128
_BPW = _B // _NW
_KALL = _BPW // _IW
_C = 256
_K = _C // _IW
_NCHUNK = _BPW // _C


def _body(wp_hbm, idx_hbm, out_hbm, idx_v, p_v, sem_g, sem_o):
    wid = lax.axis_index("s") * _NC + lax.axis_index("c")
    row0 = wid * _KALL
    base0 = wid * _BPW

    pltpu.sync_copy(idx_hbm.at[pl.ds(row0, _KALL)], idx_v)

    def fire_gathers(g, buf):
        for j in range(_K):
            pltpu.async_copy(
                wp_hbm.at[idx_v.at[g * _K + j]],
                p_v.at[pl.ds(buf * _C + j * _IW, _IW)],
                sem_g,
            )

    def drain_gathers(g, buf):
        for j in range(_K):
            pltpu.make_async_copy(
                wp_hbm.at[idx_v.at[g * _K + j]],
                p_v.at[pl.ds(buf * _C + j * _IW, _IW)],
                sem_g,
            ).wait()

    def fire_store(g, buf):
        pltpu.async_copy(
            p_v.at[pl.ds(buf * _C, _C)],
            out_hbm.at[pl.ds(base0 + g * _C, _C)],
            sem_o,
        )

    def wait_store(g, buf):
        pltpu.make_async_copy(
            p_v.at[pl.ds(buf * _C, _C)],
            out_hbm.at[pl.ds(base0 + g * _C, _C)],
            sem_o,
        ).wait()

    fire_gathers(0, 0)

    def pair(gg, carry):
        for b in range(2):
            g = gg * 2 + b
            nb = 1 - b

            @pl.when(g < _NCHUNK - 1)
            def _fill_next():
                @pl.when(g >= 1)
                def _free_buf():
                    wait_store(g - 1, nb)

                fire_gathers(g + 1, nb)

            drain_gathers(g, b)
            fire_store(g, b)
        return carry

    lax.fori_loop(0, _NCHUNK // 2, pair, 0)
    wait_store(_NCHUNK - 2, 0)
    wait_store(_NCHUNK - 1, 1)


@jax.jit
def _embedding_lookup(input_, weight):
    wp = jnp.pad(weight, ((0, 0), (0, _D)))
    idx2 = input_.astype(jnp.int32).reshape(_B // _IW, _IW)
    mesh = plsc.VectorSubcoreMesh(core_axis_name="c", subcore_axis_name="s")
    outp = pl.kernel(
        _body,
        out_type=jax.ShapeDtypeStruct((_B, _W), jnp.float32),
        mesh=mesh,
        scratch_types=[
            pltpu.VMEM((_KALL, _IW), jnp.int32),
            pltpu.VMEM((2 * _C, _W), jnp.float32),
            pltpu.SemaphoreType.DMA,
            pltpu.SemaphoreType.DMA,
        ],
        compiler_params=pltpu.CompilerParams(
            use_tc_tiling_on_sc=True,
            needs_layout_passes=False,
            skip_device_barrier=True,
        ),
    )(wp, idx2)
    return outp[:, :_D].reshape(_BATCH, _HIST, _D)


def kernel(input_, weight):
    return _embedding_lookup(input_, weight)
